# Initial kernel scaffold; baseline (speedup 1.0000x reference)
import jax
import jax.numpy as jnp
from jax.experimental import pallas as pl


def kernel(out0, out1, out2, targets):
    raise NotImplementedError("write your pallas kernel here")


# speedup vs baseline: 68.3735x; 68.3735x over previous
import functools

import jax
import jax.numpy as jnp
from jax import lax
from jax.experimental import pallas as pl

_ANCHORS = ((12.0, 16.0), (19.0, 36.0), (40.0, 28.0), (36.0, 75.0),
            (76.0, 55.0), (72.0, 146.0), (142.0, 110.0), (192.0, 243.0),
            (459.0, 401.0))
_STRIDES = (32, 16, 8)
_AMASKS = ((6, 7, 8), (3, 4, 5), (0, 1, 2))
_NCLS = 80
_NCH = 5 + _NCLS
_M = 20
_B = 16
_P = 16


def _prep_kernel(t_ref, o_ref):
    cls = t_ref[0]
    xn = t_ref[1]
    yn = t_ref[2]
    wn = t_ref[3]
    hn = t_ref[4]
    s = cls + xn + yn + wn + hn
    validrow = (s > 0.0).astype(jnp.float32)
    nlabel = jnp.sum(validrow, axis=1, keepdims=True)
    iota_m = lax.broadcasted_iota(jnp.int32, (_B, _M), 1).astype(jnp.float32)
    valid = (iota_m < nlabel).astype(jnp.float32)
    hasl = jnp.where(nlabel > 0.0, 1.0, 0.0) + jnp.zeros((_B, _M), jnp.float32)
    w5 = wn * 512.0
    h5 = hn * 512.0
    best = jnp.zeros((_B, _M), jnp.float32)
    cur = None
    for k in range(9):
        wa, ha = _ANCHORS[k]
        iw = jnp.minimum(w5, wa)
        ih = jnp.minimum(h5, ha)
        en = ((iw > 0.0) & (ih > 0.0)).astype(jnp.float32)
        ai = iw * ih * en
        iou = ai / (w5 * h5 + wa * ha - ai + 1e-16)
        if cur is None:
            cur = iou
        else:
            upd = iou > cur
            best = jnp.where(upd, float(k), best)
            cur = jnp.where(upd, iou, cur)
    a = best - 3.0 * jnp.floor(best / 3.0)
    blayer = jnp.floor(best / 3.0)
    wab = jnp.zeros_like(best)
    hab = jnp.zeros_like(best)
    for k in range(9):
        wab = jnp.where(best == float(k), _ANCHORS[k][0], wab)
        hab = jnp.where(best == float(k), _ANCHORS[k][1], hab)
    twlog = jnp.log(w5 / wab + 1e-16)
    thlog = jnp.log(h5 / hab + 1e-16)
    sc = jnp.sqrt(2.0 - wn * hn)
    o_ref[0] = valid
    o_ref[1] = hasl
    o_ref[2] = a
    o_ref[3] = blayer
    o_ref[4] = twlog
    o_ref[5] = thlog
    o_ref[6] = sc
    o_ref[7] = cls
    o_ref[8] = xn
    o_ref[9] = yn
    o_ref[10] = wn
    o_ref[11] = hn
    for l in range(3):
        f = float(512 // _STRIDES[l])
        il = jnp.floor(xn * f)
        jl = jnp.floor(yn * f)
        condl = (valid > 0.0) & (blayer == float(2 - l))
        clsact = jnp.zeros((_B, _M), jnp.float32)
        for m in range(_M):
            eq = ((a == a[:, m:m + 1]) & (il == il[:, m:m + 1]) &
                  (jl == jl[:, m:m + 1]) & (cls == cls[:, m:m + 1]))
            later = iota_m > float(m)
            dup = jnp.max(jnp.where(condl & eq & later, 1.0, 0.0),
                          axis=1, keepdims=True)
            val = jnp.where(condl[:, m:m + 1] & (dup < 0.5), 1.0, 0.0)
            clsact = jnp.where(iota_m == float(m), val, clsact)
        o_ref[12 + l] = clsact
    o_ref[15] = jnp.zeros((_B, _M), jnp.float32)


def _layer_kernel(prep_ref, x_ref, o_ref, *, lid, f, S):
    aidx = pl.program_id(1)
    stride = _STRIDES[lid]
    was = [_ANCHORS[k][0] / stride for k in _AMASKS[lid]]
    has = [_ANCHORS[k][1] / stride for k in _AMASKS[lid]]
    af = aidx.astype(jnp.float32)
    wa = jnp.where(aidx == 0, was[0], jnp.where(aidx == 1, was[1], was[2]))
    ha = jnp.where(aidx == 0, has[0], jnp.where(aidx == 1, has[1], has[2]))
    xr = x_ref[0, 0, 0]
    yr = x_ref[0, 0, 1]
    wr = x_ref[0, 0, 2]
    hr = x_ref[0, 0, 3]
    obr = x_ref[0, 0, 4]
    sx = jax.nn.sigmoid(xr)
    sy = jax.nn.sigmoid(yr)
    so = jax.nn.sigmoid(obr)
    flat = (lax.broadcasted_iota(jnp.int32, (S, 128), 0) * 128 +
            lax.broadcasted_iota(jnp.int32, (S, 128), 1))
    iif = (flat % f).astype(jnp.float32)
    jjf = (flat // f).astype(jnp.float32)
    px = sx + iif
    py = sy + jjf
    pw = jnp.exp(wr) * wa
    ph = jnp.exp(hr) * ha
    pa = pw * ph
    phw = pw * 0.5
    phh = ph * 0.5
    mx = jnp.zeros((S, 128), jnp.float32)
    assigned = jnp.zeros((S, 128), jnp.bool_)
    txf = jnp.zeros((S, 128), jnp.float32)
    tyf = jnp.zeros((S, 128), jnp.float32)
    twl = jnp.zeros((S, 128), jnp.float32)
    thl = jnp.zeros((S, 128), jnp.float32)
    scv = jnp.zeros((S, 128), jnp.float32)
    adjv = jnp.zeros((S, 128), jnp.float32)
    for m in range(_M):
        valid = prep_ref[0, 0, m] > 0.0
        am = prep_ref[0, 2, m]
        bl = prep_ref[0, 3, m]
        twlog = prep_ref[0, 4, m]
        thlog = prep_ref[0, 5, m]
        scm = prep_ref[0, 6, m]
        cm = prep_ref[0, 7, m]
        tx = prep_ref[0, 8, m] * f
        ty = prep_ref[0, 9, m] * f
        tw = prep_ref[0, 10, m] * f
        th = prep_ref[0, 11, m] * f
        clsact = prep_ref[0, 12 + lid, m] > 0.0
        hw = tw * 0.5
        hh = th * 0.5
        tlx = jnp.maximum(px - phw, tx - hw)
        brx = jnp.minimum(px + phw, tx + hw)
        tly = jnp.maximum(py - phh, ty - hh)
        bry = jnp.minimum(py + phh, ty + hh)
        en = ((tlx < brx) & (tly < bry)).astype(jnp.float32)
        ai = (brx - tlx) * (bry - tly) * en
        iou = ai / (pa + tw * th - ai + 1e-16)
        mx = jnp.maximum(mx, jnp.where(valid, iou, 0.0))
        im = jnp.floor(tx)
        jm = jnp.floor(ty)
        condm = valid & (bl == float(2 - lid)) & (am == af)
        mv = condm & (iif == im) & (jjf == jm)
        assigned = assigned | mv
        txf = jnp.where(mv, tx - im, txf)
        tyf = jnp.where(mv, ty - jm, tyf)
        twl = jnp.where(mv, twlog, twl)
        thl = jnp.where(mv, thlog, thl)
        scv = jnp.where(mv, scm, scv)
        ci = 5 + cm.astype(jnp.int32)
        vrow = x_ref[0, 0, pl.ds(ci, 1)][0]
        pc = jnp.clip(jax.nn.sigmoid(vrow), 1e-7, 1.0 - 1e-7)
        adjv = adjv + jnp.where(mv & clsact,
                                -jnp.log(pc) + jnp.log(1.0 - pc), 0.0)
    hasl = prep_ref[0, 1, 0] > 0.0
    asf = assigned.astype(jnp.float32)
    omb = jnp.where(hasl, jnp.where(mx > 0.7, 0.0, 1.0), 1.0)
    om = jnp.where(assigned, 1.0, omb)
    pobj = jnp.clip(so * om, 1e-7, 1.0 - 1e-7)
    lobj = -(asf * jnp.log(pobj) + (1.0 - asf) * jnp.log(1.0 - pobj))
    w2 = scv * scv
    pxc = jnp.clip(sx * asf, 1e-7, 1.0 - 1e-7)
    pyc = jnp.clip(sy * asf, 1e-7, 1.0 - 1e-7)
    txt = txf * asf
    tyt = tyf * asf
    lxy = (-(txt * jnp.log(pxc) + (1.0 - txt) * jnp.log(1.0 - pxc)) * w2
           - (tyt * jnp.log(pyc) + (1.0 - tyt) * jnp.log(1.0 - pyc)) * w2)
    dw = wr * asf * scv - twl * asf * scv
    dh = hr * asf * scv - thl * asf * scv
    lwh = 0.5 * (dw * dw + dh * dh)
    crows = x_ref[0, 0, pl.ds(5, _NCLS)]
    pcl = jnp.clip(jax.nn.sigmoid(crows), 1e-7, 1.0 - 1e-7)
    t1 = jnp.sum(-jnp.log(1.0 - pcl), axis=0)
    c0 = -jnp.log(1.0 - jnp.clip(jnp.float32(0.0), 1e-7, 1.0 - 1e-7))
    lcls = jnp.where(assigned, t1, _NCLS * c0) + adjv
    total = jnp.sum(lobj + lxy + lwh + lcls)
    first = (pl.program_id(0) == 0) & (pl.program_id(1) == 0)

    @pl.when(first)
    def _():
        o_ref[...] = jnp.zeros_like(o_ref)

    ri = lax.broadcasted_iota(jnp.int32, (8, 128), 0)
    ci = lax.broadcasted_iota(jnp.int32, (8, 128), 1)
    o_ref[...] += jnp.where((ri == 0) & (ci == 0), total, 0.0)


def _run_layer(prep, xresh, lid, f, S):
    kern = functools.partial(_layer_kernel, lid=lid, f=f, S=S)
    return pl.pallas_call(
        kern,
        grid=(_B, 3),
        in_specs=[
            pl.BlockSpec((1, _P, _M), lambda b, a: (b, 0, 0)),
            pl.BlockSpec((1, 1, _NCH, S, 128), lambda b, a: (b, a, 0, 0, 0)),
        ],
        out_specs=pl.BlockSpec((8, 128), lambda b, a: (0, 0)),
        out_shape=jax.ShapeDtypeStruct((8, 128), jnp.float32),
    )(prep, xresh)


def kernel(out0, out1, out2, targets):
    tgt_t = jnp.transpose(targets, (2, 0, 1))
    prep = pl.pallas_call(
        _prep_kernel,
        out_shape=jax.ShapeDtypeStruct((_P, _B, _M), jnp.float32),
    )(tgt_t)
    prep = jnp.transpose(prep, (1, 0, 2))
    total = jnp.float32(0.0)
    for lid, out in enumerate((out0, out1, out2)):
        f = out.shape[2]
        S = f * f // 128
        xresh = out.reshape(_B, 3, _NCH, S, 128)
        total = total + _run_layer(prep, xresh, lid, f, S)[0, 0]
    return total
